# baseline (speedup 1.0000x reference)
import functools

import jax
import jax.numpy as jnp
from jax.experimental import pallas as pl
from jax.experimental.pallas import tpu as pltpu
from jax.experimental.pallas import tpu_sc as plsc

T = 2048
D = 1024
FF = 4096
E = 8
K = 2
BK = 256
NB = 24
P = T * K
CH = 512
NF = 4
JB = 8
NBT = NB
SLOTS = NB * BK
SC_NC = 2
SC_NS = 16
NW = SC_NC * SC_NS
PPW = P // NW
CHR = 32
NCH = PPW // CHR

_f32 = jnp.float32
_bf16 = jnp.bfloat16
_i32 = jnp.int32


def _dot(a, b):
    return jax.lax.dot_general(a, b, (((1,), (0,)), ((), ())),
                               preferred_element_type=_f32)


def _router_body(x_ref, gw_ref, logits_ref, pos_ref, wgt_ref, meta_ref):
    x = x_ref[...]
    logits = jax.lax.dot_general(
        x, gw_ref[...], (((1,), (0,)), ((), ())),
        preferred_element_type=_f32, precision=jax.lax.Precision.DEFAULT)
    logits_ref[...] = logits

    m = jnp.max(logits, axis=1, keepdims=True)
    p = jnp.exp(logits - m)
    probs = p / jnp.sum(p, axis=1, keepdims=True)

    bw = probs[:, 0:1]
    bi = jnp.zeros((T, 1), _i32)
    for e in range(1, E):
        c = probs[:, e:e + 1]
        upd = c > bw
        bi = jnp.where(upd, e, bi)
        bw = jnp.where(upd, c, bw)
    sw = jnp.full((T, 1), -1.0, _f32)
    si = jnp.zeros((T, 1), _i32)
    for e in range(E):
        c = probs[:, e:e + 1]
        upd = jnp.logical_and(bi != e, c > sw)
        si = jnp.where(upd, e, si)
        sw = jnp.where(upd, c, sw)
    tot = bw + sw
    w0 = bw / tot
    w1v = sw / tot

    er = jax.lax.broadcasted_iota(_i32, (1, E), 1)
    oh0 = (bi == er).astype(_f32)
    oh1 = (si == er).astype(_f32)
    counts = jnp.sum(oh0 + oh1, axis=0, keepdims=True)
    blocks = jnp.ceil(counts * (1.0 / BK))
    u_strict = (jax.lax.broadcasted_iota(_i32, (E, E), 0)
                < jax.lax.broadcasted_iota(_i32, (E, E), 1)).astype(_f32)
    start = _dot(blocks, u_strict)
    slot_off = start * BK
    nact = start[:, E - 1:E] + blocks[:, E - 1:E]

    ohcat = jnp.concatenate([oh0, oh1], axis=0)
    offcat = jnp.sum(ohcat * slot_off, axis=1, keepdims=True)
    ltri = (jax.lax.broadcasted_iota(_i32, (CH, CH), 0)
            > jax.lax.broadcasted_iota(_i32, (CH, CH), 1)).astype(_bf16)
    carry = jnp.zeros((1, E), _f32)
    ranks = []
    for c in range(P // CH):
        oc = ohcat[c * CH:(c + 1) * CH]
        within = _dot(ltri, oc.astype(_bf16)) + carry
        ranks.append(jnp.sum(oc * within, axis=1, keepdims=True))
        carry = carry + jnp.sum(oc, axis=0, keepdims=True)
    rankv = jnp.concatenate(ranks, axis=0)
    pos_ref[...] = (offcat + rankv).astype(_i32)
    wgt_ref[...] = jnp.concatenate([w0, w1v], axis=0)

    jm = jax.lax.broadcasted_iota(_i32, (1, 32), 1)
    startb = jnp.concatenate([start, jnp.zeros((1, 32 - E), _f32)], axis=1)
    blocksb = jnp.concatenate(
        [jnp.zeros((1, E), _f32), blocks, jnp.zeros((1, 32 - 2 * E), _f32)],
        axis=1)
    countsb = jnp.concatenate(
        [jnp.zeros((1, 2 * E), _f32), counts, jnp.zeros((1, 32 - 3 * E), _f32)],
        axis=1)
    meta_ref[...] = jnp.where(
        jm < E, startb, jnp.where(jm < 2 * E, blocksb, countsb)).astype(_i32)


def _dispatch_body(x_hbm, tok_hbm, pos_hbm, out_hbm, tok_v, pos_v, rows_v, sem):
    wid = jax.lax.axis_index("s") * SC_NC + jax.lax.axis_index("c")
    pltpu.sync_copy(tok_hbm.at[wid], tok_v)
    pltpu.sync_copy(pos_hbm.at[wid], pos_v)
    for c in range(NCH):
        pltpu.async_copy(x_hbm.at[tok_v.at[c]], rows_v, sem).wait()
        pltpu.async_copy(rows_v, out_hbm.at[pos_v.at[c]], sem).wait()


def _mlp_body(meta_ref, xsw_ref, posr_ref, wgtr_ref,
              w1_ref, w3_ref, w2_ref, out_ref,
              xs_ref, yacc_ref, w1b_ref, w3b_ref, w2b_ref):
    e = pl.program_id(0)
    f = pl.program_id(1)
    jb = pl.program_id(2)

    @pl.when(jnp.logical_and(e == 0, jnp.logical_and(f == 0, jb == 0)))
    def _init():
        out_ref[...] = jnp.zeros_like(out_ref)

    @pl.when(jb == 0)
    def _cast():
        w1b_ref[...] = w1_ref[0].astype(_bf16)
        w3b_ref[...] = w3_ref[0].astype(_bf16)
        w2b_ref[...] = w2_ref[0].astype(_bf16)

    @pl.when(jb < meta_ref[E + e])
    def _compute():
        base = (meta_ref[e] + jb) * BK
        posr = posr_ref[...]
        ii = jax.lax.broadcasted_iota(_i32, (BK, T), 0) + base

        @pl.when(f == 0)
        def _gather():
            xs_ref[pl.ds(jb, 1)] = xsw_ref[...].astype(_bf16)

        xs = xs_ref[pl.ds(jb, 1)][0]
        a = _dot(xs, w1b_ref[...])
        bv = _dot(xs, w3b_ref[...])
        h = (a * jax.nn.sigmoid(a) * bv).astype(_bf16)
        yp = _dot(h, w2b_ref[...])

        @pl.when(f == 0)
        def _y0():
            yacc_ref[pl.ds(jb, 1)] = yp[None]

        @pl.when(f > 0)
        def _y1():
            yacc_ref[pl.ds(jb, 1)] += yp[None]

        @pl.when(f == NF - 1)
        def _scatter():
            wgtr = wgtr_ref[...]
            st = (jnp.where(posr[0:1, :] == ii, wgtr[0:1, :], 0.0)
                  + jnp.where(posr[1:2, :] == ii, wgtr[1:2, :], 0.0)
                  ).astype(_bf16)
            rem = meta_ref[2 * E + e] - jb * BK
            valid = jax.lax.broadcasted_iota(_i32, (BK, 1), 0) < rem
            yv = jnp.where(valid, yacc_ref[pl.ds(jb, 1)][0], 0.0)
            out_ref[...] += jax.lax.dot_general(
                st, yv.astype(_bf16),
                (((0,), (0,)), ((), ())),
                preferred_element_type=_f32)


def _moe(hidden_states, gate_w, w1, w3, w2, interpret=False):
    b, s, d = hidden_states.shape
    x = hidden_states.reshape(T, D)

    logits, posp, wgtp, bmeta = pl.pallas_call(
        _router_body,
        out_shape=(
            jax.ShapeDtypeStruct((T, E), _f32),
            jax.ShapeDtypeStruct((P, 1), _i32),
            jax.ShapeDtypeStruct((P, 1), _f32),
            jax.ShapeDtypeStruct((1, 32), _i32),
        ),
        interpret=interpret,
    )(x, gate_w)

    meta = bmeta.reshape(32)
    pos_r = posp.reshape(K, T)
    wgt_r = wgtp.reshape(K, T)

    tok_ids = (jnp.arange(P, dtype=_i32) % T).reshape(NW, NCH, CHR)
    pos_sc = posp.reshape(NW, NCH, CHR)
    sc_kernel = functools.partial(
        pl.kernel,
        mesh=plsc.VectorSubcoreMesh(core_axis_name="c", subcore_axis_name="s"),
        out_type=jax.ShapeDtypeStruct((SLOTS, D), _f32),
        scratch_types=[
            pltpu.VMEM((NCH, CHR), _i32),
            pltpu.VMEM((NCH, CHR), _i32),
            pltpu.VMEM((CHR, D), _f32),
            pltpu.SemaphoreType.DMA,
        ],
    )(_dispatch_body)
    xs_sorted = sc_kernel(x, tok_ids, pos_sc).reshape(NBT, BK, D)

    fh = FF // NF

    def _xs_idx(e, f, jb, m):
        jx = jnp.where(f == 0, m[e] + jb, m[e] + JB - 1)
        return (jnp.minimum(jx, NBT - 1), 0, 0)

    grid_spec = pltpu.PrefetchScalarGridSpec(
        num_scalar_prefetch=1,
        grid=(E, NF, JB),
        in_specs=[
            pl.BlockSpec((1, BK, D), _xs_idx),
            pl.BlockSpec((K, T), lambda e, f, jb, m: (0, 0)),
            pl.BlockSpec((K, T), lambda e, f, jb, m: (0, 0)),
            pl.BlockSpec((1, D, fh), lambda e, f, jb, m: (e, 0, f)),
            pl.BlockSpec((1, D, fh), lambda e, f, jb, m: (e, 0, f)),
            pl.BlockSpec((1, fh, D), lambda e, f, jb, m: (e, f, 0)),
        ],
        out_specs=pl.BlockSpec((T, D), lambda e, f, jb, m: (0, 0)),
        scratch_shapes=[
            pltpu.VMEM((JB, BK, D), _bf16),
            pltpu.VMEM((JB, BK, D), _f32),
            pltpu.VMEM((D, fh), _bf16),
            pltpu.VMEM((D, fh), _bf16),
            pltpu.VMEM((fh, D), _bf16),
        ],
    )
    final = pl.pallas_call(
        _mlp_body,
        grid_spec=grid_spec,
        out_shape=jax.ShapeDtypeStruct((T, D), _f32),
        interpret=interpret,
    )(meta, xs_sorted, pos_r, wgt_r, w1, w3, w2)

    return final.reshape(b, s, d), logits


def kernel(hidden_states, gate_w, w1, w3, w2):
    return _moe(hidden_states, gate_w, w1, w3, w2)
